# baseline (speedup 1.0000x reference)
import functools

import jax
import jax.numpy as jnp
from jax import lax
from jax.experimental import pallas as pl
from jax.experimental.pallas import tpu as pltpu
from jax.experimental.pallas import tpu_sc as plsc

_B = 4
_S = 2048
_D = 768
_EPS = 1e-12

_N_TOK = _B * _S
_NHALF = _N_TOK // 2
_NW = 32
_TPW = _NHALF // _NW
_K = 64
_NCH = _TPW // _K

_BLK_T = 2048


def _gather_body(ids_hbm, wemb_hbm, out_hbm,
                 idx0, idx1, row0, row1, sg0, sg1, ss0, ss1):
    cid = lax.axis_index("c")
    sid = lax.axis_index("s")
    base = (sid * 2 + cid) * _TPW
    idx = (idx0, idx1)
    row = (row0, row1)
    sg = (sg0, sg1)
    ss = (ss0, ss1)

    pltpu.sync_copy(ids_hbm.at[pl.ds(base, _K)], idx0)
    pltpu.async_copy(wemb_hbm.at[idx0], row0, sg0)
    for c in range(_NCH):
        b = c & 1
        if c + 1 < _NCH:
            pltpu.sync_copy(ids_hbm.at[pl.ds(base + (c + 1) * _K, _K)],
                            idx[1 - b])
            if c >= 1:
                pltpu.make_async_copy(
                    row[1 - b], out_hbm.at[pl.ds(base + (c - 1) * _K, _K)],
                    ss[1 - b]).wait()
            pltpu.async_copy(wemb_hbm.at[idx[1 - b]], row[1 - b], sg[1 - b])
        pltpu.make_async_copy(wemb_hbm.at[idx[b]], row[b], sg[b]).wait()
        pltpu.async_copy(row[b], out_hbm.at[pl.ds(base + c * _K, _K)], ss[b])
    for c in (_NCH - 2, _NCH - 1):
        b = c & 1
        pltpu.make_async_copy(
            row[b], out_hbm.at[pl.ds(base + c * _K, _K)], ss[b]).wait()


def _sc_gather(ids, wemb):
    mesh = plsc.VectorSubcoreMesh(core_axis_name="c", subcore_axis_name="s")
    f = functools.partial(
        pl.kernel,
        mesh=mesh,
        compiler_params=pltpu.CompilerParams(needs_layout_passes=False),
        out_type=jax.ShapeDtypeStruct((_NHALF, _D), jnp.float32),
        scratch_types=[
            pltpu.VMEM((_K,), jnp.int32),
            pltpu.VMEM((_K,), jnp.int32),
            pltpu.VMEM((_K, _D), jnp.float32),
            pltpu.VMEM((_K, _D), jnp.float32),
            pltpu.SemaphoreType.DMA,
            pltpu.SemaphoreType.DMA,
            pltpu.SemaphoreType.DMA,
            pltpu.SemaphoreType.DMA,
        ],
    )(_gather_body)
    return f(ids, wemb)


def _ln_body(rows_ref, pos_ref, typ_ref, gam_ref, bet_ref, out_ref):
    x = rows_ref[...] + pos_ref[...] + typ_ref[...]
    mean = jnp.mean(x, axis=1, keepdims=True)
    xc = x - mean
    var = jnp.mean(xc * xc, axis=1, keepdims=True)
    y = xc * lax.rsqrt(var + _EPS)
    out_ref[...] = (y * gam_ref[...] + bet_ref[...]).astype(jnp.bfloat16)


def _tc_layernorm(rows, pos, typ0, gam2, bet2):
    nh = _S // _BLK_T
    nb = _NHALF // _S
    return pl.pallas_call(
        _ln_body,
        grid=(nh, nb),
        in_specs=[
            pl.BlockSpec((_BLK_T, _D), lambda h, b: (b * nh + h, 0)),
            pl.BlockSpec((_BLK_T, _D), lambda h, b: (h, 0)),
            pl.BlockSpec((1, _D), lambda h, b: (0, 0)),
            pl.BlockSpec((1, _D), lambda h, b: (0, 0)),
            pl.BlockSpec((1, _D), lambda h, b: (0, 0)),
        ],
        out_specs=pl.BlockSpec((_BLK_T, _D), lambda h, b: (b * nh + h, 0)),
        out_shape=jax.ShapeDtypeStruct((_NHALF, _D), jnp.bfloat16),
    )(rows, pos, typ0, gam2, bet2)


@jax.jit
def _run(ids, wemb, pemb, temb, gam, bet):
    typ0 = temb[0:1]
    gam2 = gam.reshape(1, _D)
    bet2 = bet.reshape(1, _D)
    pos = pemb[:_S]
    rows_lo = _sc_gather(ids[:_NHALF], wemb)
    rows_hi = _sc_gather(ids[_NHALF:], wemb)
    out_lo = _tc_layernorm(rows_lo, pos, typ0, gam2, bet2)
    out_hi = _tc_layernorm(rows_hi, pos, typ0, gam2, bet2)
    return lax.concatenate([out_lo, out_hi], 0)


def kernel(input_ids, word_emb, pos_emb, type_emb, gamma, beta):
    b, s = input_ids.shape
    ids = input_ids.reshape(-1).astype(jnp.int32)
    out = _run(ids, word_emb, pos_emb, type_emb, gamma, beta)
    return out.reshape(b, s, _D)


# speedup vs baseline: 1.1984x; 1.1984x over previous
import functools

import jax
import jax.numpy as jnp
from jax import lax
from jax.experimental import pallas as pl
from jax.experimental.pallas import tpu as pltpu
from jax.experimental.pallas import tpu_sc as plsc

_B = 4
_S = 2048
_D = 768
_EPS = 1e-12

_N_TOK = _B * _S
_NW = 32
_TPW = _N_TOK // _NW
_K = 64
_NCH = _TPW // _K

_BLK_T = 2048


def _gather_body(ids_hbm, wemb_hbm, out_hbm,
                 idx0, idx1, row0, row1, sg0, sg1, ss0, ss1):
    cid = lax.axis_index("c")
    sid = lax.axis_index("s")
    base = (sid * 2 + cid) * _TPW
    idx = (idx0, idx1)
    row = (row0, row1)
    sg = (sg0, sg1)
    ss = (ss0, ss1)

    pltpu.sync_copy(ids_hbm.at[pl.ds(base, _K)], idx0)
    pltpu.async_copy(wemb_hbm.at[idx0], row0, sg0)
    for c in range(_NCH):
        b = c & 1
        if c + 1 < _NCH:
            pltpu.sync_copy(ids_hbm.at[pl.ds(base + (c + 1) * _K, _K)],
                            idx[1 - b])
            if c >= 1:
                pltpu.make_async_copy(
                    row[1 - b], out_hbm.at[pl.ds(base + (c - 1) * _K, _K)],
                    ss[1 - b]).wait()
            pltpu.async_copy(wemb_hbm.at[idx[1 - b]], row[1 - b], sg[1 - b])
        pltpu.make_async_copy(wemb_hbm.at[idx[b]], row[b], sg[b]).wait()
        pltpu.async_copy(row[b], out_hbm.at[pl.ds(base + c * _K, _K)], ss[b])
    for c in (_NCH - 2, _NCH - 1):
        b = c & 1
        pltpu.make_async_copy(
            row[b], out_hbm.at[pl.ds(base + c * _K, _K)], ss[b]).wait()


def _sc_gather(ids, wemb):
    mesh = plsc.VectorSubcoreMesh(core_axis_name="c", subcore_axis_name="s")
    f = functools.partial(
        pl.kernel,
        mesh=mesh,
        compiler_params=pltpu.CompilerParams(needs_layout_passes=False),
        out_type=jax.ShapeDtypeStruct((_N_TOK, _D), jnp.float32),
        scratch_types=[
            pltpu.VMEM((_K,), jnp.int32),
            pltpu.VMEM((_K,), jnp.int32),
            pltpu.VMEM((_K, _D), jnp.float32),
            pltpu.VMEM((_K, _D), jnp.float32),
            pltpu.SemaphoreType.DMA,
            pltpu.SemaphoreType.DMA,
            pltpu.SemaphoreType.DMA,
            pltpu.SemaphoreType.DMA,
        ],
    )(_gather_body)
    return f(ids, wemb)


def _ln_body(rows_ref, pos_ref, typ_ref, gam_ref, bet_ref, out_ref):
    x = rows_ref[...] + pos_ref[...] + typ_ref[...]
    mean = jnp.mean(x, axis=1, keepdims=True)
    xc = x - mean
    var = jnp.mean(xc * xc, axis=1, keepdims=True)
    y = xc * lax.rsqrt(var + _EPS)
    out_ref[...] = (y * gam_ref[...] + bet_ref[...]).astype(jnp.bfloat16)


def _tc_layernorm(rows, pos, typ0, gam2, bet2):
    nh = _S // _BLK_T
    nb = _N_TOK // _S
    return pl.pallas_call(
        _ln_body,
        grid=(nb, nh),
        in_specs=[
            pl.BlockSpec((_BLK_T, _D), lambda b, h: (b * nh + h, 0)),
            pl.BlockSpec((_BLK_T, _D), lambda b, h: (h, 0)),
            pl.BlockSpec((1, _D), lambda b, h: (0, 0)),
            pl.BlockSpec((1, _D), lambda b, h: (0, 0)),
            pl.BlockSpec((1, _D), lambda b, h: (0, 0)),
        ],
        out_specs=pl.BlockSpec((_BLK_T, _D), lambda b, h: (b * nh + h, 0)),
        out_shape=jax.ShapeDtypeStruct((_N_TOK, _D), jnp.bfloat16),
    )(rows, pos, typ0, gam2, bet2)


@jax.jit
def _run(ids, wemb, pemb, temb, gam, bet):
    typ0 = temb[0:1]
    gam2 = gam.reshape(1, _D)
    bet2 = bet.reshape(1, _D)
    pos = pemb[:_S]
    rows = _sc_gather(ids, wemb)
    return _tc_layernorm(rows, pos, typ0, gam2, bet2)


def kernel(input_ids, word_emb, pos_emb, type_emb, gamma, beta):
    b, s = input_ids.shape
    ids = input_ids.reshape(-1).astype(jnp.int32)
    out = _run(ids, word_emb, pos_emb, type_emb, gamma, beta)
    return out.reshape(b, s, _D)
